# baseline (speedup 1.0000x reference)
import functools

import jax
import jax.numpy as jnp
from jax import lax
from jax.experimental import pallas as pl
from jax.experimental.pallas import tpu as pltpu
from jax.experimental.pallas import tpu_sc as plsc

B = 16384
V = 1000000
D = 64

NC = 2
NS = 16
NW = NC * NS
BPW = B // NW
FK = 16


def _gather3_body(rt_hbm, re_hbm, rm_hbm, e1_hbm, e2_hbm, e3_hbm,
                  g1_hbm, g2_hbm, g3_hbm, idx_v, rows_v, sem, sem_out):
    wid = lax.axis_index("s") * NC + lax.axis_index("c")
    base = wid * BPW
    idx_hbms = (rt_hbm, re_hbm, rm_hbm)
    tab_hbms = (e1_hbm, e2_hbm, e3_hbm)
    out_hbms = (g1_hbm, g2_hbm, g3_hbm)
    for t in range(3):
        pltpu.sync_copy(idx_hbms[t].at[pl.ds(base, BPW)],
                        idx_v.at[pl.ds(t * BPW, BPW)])

    def do_table(t, tab, out_hbm):
        def fire(g, _):
            idx16 = idx_v[pl.ds(t * BPW + g * FK, FK)]
            for u in range(FK):
                j = g * FK + u
                pltpu.async_copy(tab.at[pl.ds(idx16[u], 1)],
                                 rows_v.at[pl.ds(j, 1)], sem)
            for u in range(FK):
                pltpu.make_async_copy(tab.at[pl.ds(0, 1)],
                                      rows_v.at[pl.ds(g * FK + u, 1)],
                                      sem).wait()
            return ()

        lax.fori_loop(0, BPW // FK, fire, (), unroll=False)
        pltpu.async_copy(rows_v, out_hbm.at[pl.ds(base, BPW)], sem_out).wait()

    for t in range(3):
        do_table(t, tab_hbms[t], out_hbms[t])


_gather3 = pl.kernel(
    _gather3_body,
    out_type=(jax.ShapeDtypeStruct((B, D), jnp.float32),) * 3,
    mesh=plsc.VectorSubcoreMesh(core_axis_name="c", subcore_axis_name="s",
                                num_cores=NC, num_subcores=NS),
    scratch_types=[
        pltpu.VMEM((3 * BPW,), jnp.int32),
        pltpu.VMEM((BPW, D), jnp.float32),
        pltpu.SemaphoreType.DMA,
        pltpu.SemaphoreType.DMA,
    ],
)


BM = 2048


def _mm_body(g1, g2, g3, wt, bb, out):
    acc = jnp.dot(g1[...], wt[0:D, :], preferred_element_type=jnp.float32)
    acc += jnp.dot(g2[...], wt[D:2 * D, :], preferred_element_type=jnp.float32)
    acc += jnp.dot(g3[...], wt[2 * D:3 * D, :], preferred_element_type=jnp.float32)
    acc += bb[...]
    out[...] = jnp.maximum(acc, 0.0)


def _mm(g1, g2, g3, wt, bb):
    return pl.pallas_call(
        _mm_body,
        grid=(B // BM,),
        in_specs=[
            pl.BlockSpec((BM, D), lambda i: (i, 0)),
            pl.BlockSpec((BM, D), lambda i: (i, 0)),
            pl.BlockSpec((BM, D), lambda i: (i, 0)),
            pl.BlockSpec((3 * D, D), lambda i: (0, 0)),
            pl.BlockSpec((1, D), lambda i: (0, 0)),
        ],
        out_specs=pl.BlockSpec((BM, D), lambda i: (i, 0)),
        out_shape=jax.ShapeDtypeStruct((B, D), jnp.float32),
    )(g1, g2, g3, wt, bb)


def kernel(rt, re, rm, emb1, emb2, emb3, W, b):
    g1, g2, g3 = _gather3(rt, re, rm, emb1, emb2, emb3)
    wt = W.T
    bb = b.reshape(1, D)
    return _mm(g1, g2, g3, wt, bb)


# speedup vs baseline: 1.0475x; 1.0475x over previous
import functools

import jax
import jax.numpy as jnp
from jax import lax
from jax.experimental import pallas as pl
from jax.experimental.pallas import tpu as pltpu
from jax.experimental.pallas import tpu_sc as plsc

B = 16384
V = 1000000
D = 64

NC = 2
NS = 16
NW = NC * NS
BPW = B // NW
FK = 16


def _gather3_body(rt_hbm, re_hbm, rm_hbm, e1_hbm, e2_hbm, e3_hbm,
                  g1_hbm, g2_hbm, g3_hbm, idx_v, rows_v, sem, sem_out):
    wid = lax.axis_index("s") * NC + lax.axis_index("c")
    base = wid * BPW
    idx_hbms = (rt_hbm, re_hbm, rm_hbm)
    tab_hbms = (e1_hbm, e2_hbm, e3_hbm)
    out_hbms = (g1_hbm, g2_hbm, g3_hbm)
    for t in range(3):
        pltpu.sync_copy(idx_hbms[t].at[pl.ds(base, BPW)],
                        idx_v.at[pl.ds(t * BPW, BPW)])

    def do_table(t, tab, out_hbm):
        def fire(g, _):
            idx16 = idx_v[pl.ds(t * BPW + g * FK, FK)]
            for u in range(FK):
                j = g * FK + u
                pltpu.async_copy(tab.at[pl.ds(idx16[u], 1)],
                                 rows_v.at[pl.ds(j, 1)], sem)
            return ()

        lax.fori_loop(0, BPW // FK, fire, (), unroll=False)
        pltpu.make_async_copy(tab.at[pl.ds(0, BPW)], rows_v, sem).wait()
        return pltpu.async_copy(rows_v, out_hbm.at[pl.ds(base, BPW)], sem_out)

    for t in range(3):
        out_copy = do_table(t, tab_hbms[t], out_hbms[t])
        if t < 2:
            out_copy.wait()
    out_copy.wait()


_gather3 = pl.kernel(
    _gather3_body,
    out_type=(jax.ShapeDtypeStruct((B, D), jnp.float32),) * 3,
    mesh=plsc.VectorSubcoreMesh(core_axis_name="c", subcore_axis_name="s",
                                num_cores=NC, num_subcores=NS),
    scratch_types=[
        pltpu.VMEM((3 * BPW,), jnp.int32),
        pltpu.VMEM((BPW, D), jnp.float32),
        pltpu.SemaphoreType.DMA,
        pltpu.SemaphoreType.DMA,
    ],
)


BM = 2048


def _mm_body(g1, g2, g3, wt, bb, out):
    acc = jnp.dot(g1[...], wt[0:D, :], preferred_element_type=jnp.float32)
    acc += jnp.dot(g2[...], wt[D:2 * D, :], preferred_element_type=jnp.float32)
    acc += jnp.dot(g3[...], wt[2 * D:3 * D, :], preferred_element_type=jnp.float32)
    acc += bb[...]
    out[...] = jnp.maximum(acc, 0.0)


def _mm(g1, g2, g3, wt, bb):
    return pl.pallas_call(
        _mm_body,
        grid=(B // BM,),
        in_specs=[
            pl.BlockSpec((BM, D), lambda i: (i, 0)),
            pl.BlockSpec((BM, D), lambda i: (i, 0)),
            pl.BlockSpec((BM, D), lambda i: (i, 0)),
            pl.BlockSpec((3 * D, D), lambda i: (0, 0)),
            pl.BlockSpec((1, D), lambda i: (0, 0)),
        ],
        out_specs=pl.BlockSpec((BM, D), lambda i: (i, 0)),
        out_shape=jax.ShapeDtypeStruct((B, D), jnp.float32),
    )(g1, g2, g3, wt, bb)


def kernel(rt, re, rm, emb1, emb2, emb3, W, b):
    g1, g2, g3 = _gather3(rt, re, rm, emb1, emb2, emb3)
    wt = W.T
    bb = b.reshape(1, D)
    return _mm(g1, g2, g3, wt, bb)


# speedup vs baseline: 1.5832x; 1.5114x over previous
import functools

import jax
import jax.numpy as jnp
from jax import lax
from jax.experimental import pallas as pl
from jax.experimental.pallas import tpu as pltpu
from jax.experimental.pallas import tpu_sc as plsc

B = 16384
V = 1000000
D = 64

NC = 2
NS = 16
NW = NC * NS
BPW = B // NW
FK = 16


def _gather3_body(rt_hbm, re_hbm, rm_hbm, e1_hbm, e2_hbm, e3_hbm,
                  g1_hbm, g2_hbm, g3_hbm, idx_v, rows_v, sem, sem_out):
    wid = lax.axis_index("s") * NC + lax.axis_index("c")
    base = wid * BPW
    idx_hbms = (rt_hbm, re_hbm, rm_hbm)
    tab_hbms = (e1_hbm, e2_hbm, e3_hbm)
    out_hbms = (g1_hbm, g2_hbm, g3_hbm)
    for t in range(3):
        pltpu.sync_copy(idx_hbms[t].at[pl.ds(base, BPW)],
                        idx_v.at[pl.ds(t * BPW, BPW)])

    def do_table(t, tab, out_hbm):
        def fire(g, _):
            idx16 = idx_v[pl.ds(t * BPW + g * FK, FK)]
            for u in range(FK):
                j = g * FK + u
                i = idx16[u]
                pltpu.async_copy(tab.at[i // 8, pl.ds(i % 8, 1)],
                                 rows_v.at[j // 8, pl.ds(j % 8, 1)], sem)
            return ()

        lax.fori_loop(0, BPW // FK, fire, (), unroll=False)
        pltpu.make_async_copy(tab.at[pl.ds(0, BPW // 8)], rows_v, sem).wait()
        return pltpu.async_copy(rows_v, out_hbm.at[pl.ds(base // 8, BPW // 8)],
                                sem_out)

    for t in range(3):
        out_copy = do_table(t, tab_hbms[t], out_hbms[t])
        if t < 2:
            out_copy.wait()
    out_copy.wait()


_gather3 = pl.kernel(
    _gather3_body,
    out_type=(jax.ShapeDtypeStruct((B // 8, 8, D), jnp.float32),) * 3,
    mesh=plsc.VectorSubcoreMesh(core_axis_name="c", subcore_axis_name="s",
                                num_cores=NC, num_subcores=NS),
    scratch_types=[
        pltpu.VMEM((3 * BPW,), jnp.int32),
        pltpu.VMEM((BPW // 8, 8, D), jnp.float32),
        pltpu.SemaphoreType.DMA,
        pltpu.SemaphoreType.DMA,
    ],
)


BM = 2048


def _mm_body(g1, g2, g3, wt, bb, out):
    acc = jnp.dot(g1[...], wt[0:D, :], preferred_element_type=jnp.float32)
    acc += jnp.dot(g2[...], wt[D:2 * D, :], preferred_element_type=jnp.float32)
    acc += jnp.dot(g3[...], wt[2 * D:3 * D, :], preferred_element_type=jnp.float32)
    acc += bb[...]
    out[...] = jnp.maximum(acc, 0.0)


def _mm(g1, g2, g3, wt, bb):
    return pl.pallas_call(
        _mm_body,
        grid=(B // BM,),
        in_specs=[
            pl.BlockSpec((BM, D), lambda i: (i, 0)),
            pl.BlockSpec((BM, D), lambda i: (i, 0)),
            pl.BlockSpec((BM, D), lambda i: (i, 0)),
            pl.BlockSpec((3 * D, D), lambda i: (0, 0)),
            pl.BlockSpec((1, D), lambda i: (0, 0)),
        ],
        out_specs=pl.BlockSpec((BM, D), lambda i: (i, 0)),
        out_shape=jax.ShapeDtypeStruct((B, D), jnp.float32),
    )(g1, g2, g3, wt, bb)


def kernel(rt, re, rm, emb1, emb2, emb3, W, b):
    t1 = emb1.reshape(V // 8, 8, D)
    t2 = emb2.reshape(V // 8, 8, D)
    t3 = emb3.reshape(V // 8, 8, D)
    g1, g2, g3 = _gather3(rt, re, rm, t1, t2, t3)
    wt = W.T
    bb = b.reshape(1, D)
    return _mm(g1.reshape(B, D), g2.reshape(B, D), g3.reshape(B, D), wt, bb)
